# baseline (speedup 1.0000x reference)
import dataclasses
import functools

import jax
import jax.numpy as jnp
from jax import lax
from jax.experimental import pallas as pl
from jax.experimental.pallas import tpu as pltpu
from jax.experimental.pallas import tpu_sc as plsc

IMG_TOKEN = 151655

NC = 2
NS = 16
NW = NC * NS
LANES = 16


def _sc_combine(ids_flat, table, image):
    n = ids_flat.shape[0]
    d = table.shape[1]
    n_img = image.shape[0]
    per_w = n // NW
    chunk = 16
    mesh = plsc.VectorSubcoreMesh(core_axis_name="c", subcore_axis_name="s")
    cp = pltpu.CompilerParams()
    if "needs_layout_passes" in pltpu.CompilerParams.__dataclass_fields__:
        cp = dataclasses.replace(cp, needs_layout_passes=False)

    @functools.partial(
        pl.kernel,
        out_type=jax.ShapeDtypeStruct((n, d), jnp.float32),
        mesh=mesh,
        compiler_params=cp,
        scratch_types=[
            pltpu.VMEM((n,), jnp.int32),
            pltpu.VMEM((LANES,), jnp.int32),
            pltpu.VMEM((chunk, d), jnp.float32),
            pltpu.VMEM((chunk, d), jnp.float32),
            pltpu.VMEM((chunk, d), jnp.float32),
            pltpu.SemaphoreType.DMA,
            pltpu.SemaphoreType.DMA,
            pltpu.SemaphoreType.DMA,
            pltpu.SemaphoreType.DMA,
            pltpu.SemaphoreType.DMA,
            pltpu.SemaphoreType.DMA,
            pltpu.SemaphoreType.DMA,
        ],
    )
    def k(ids_hbm, table_hbm, img_hbm, out_hbm,
          ids_v, cnt_v, rows0, rows1, rows2,
          gsem0, gsem1, gsem2, wsem0, wsem1, wsem2, fsem):
        wid = lax.axis_index("s") * NC + lax.axis_index("c")
        base = wid * per_w
        pltpu.sync_copy(ids_hbm, ids_v)
        bufs = ((rows0, gsem0, wsem0), (rows1, gsem1, wsem1),
                (rows2, gsem2, wsem2))

        def gather_start(c, buf, gs):
            pltpu.async_copy(
                table_hbm.at[ids_v.at[pl.ds(base + c, chunk)]], buf, gs)

        def gather_wait(buf, gs):
            pltpu.make_async_copy(
                table_hbm.at[ids_v.at[pl.ds(0, chunk)]], buf, gs).wait()

        def write_wait(buf, ws):
            pltpu.make_async_copy(buf, out_hbm.at[pl.ds(0, chunk)], ws).wait()

        gather_start(0, rows0, gsem0)
        gather_start(chunk, rows1, gsem1)
        gather_start(2 * chunk, rows2, gsem2)

        cnt_v[...] = jnp.zeros((LANES,), jnp.int32)

        @pl.loop(0, base, step=LANES)
        def _(g):
            mv = ids_v[pl.ds(g, LANES)] == IMG_TOKEN
            cnt_v[...] = cnt_v[...] + plsc.all_reduce_population_count(mv)

        def fixup(c, buf):
            mvec = ids_v[pl.ds(base + c, chunk)] == IMG_TOKEN
            cs = plsc.cumsum(mvec.astype(jnp.int32))
            tot = jnp.max(cs)
            cntv = cnt_v[...]
            fvec = jnp.where(
                mvec, jnp.clip(cntv + cs - 1, 0, n_img - 1), -1)
            cnt_v[...] = cntv + tot

            @pl.when(tot > 0)
            def _():
                lanes = lax.iota(jnp.int32, chunk)
                for j in range(chunk):
                    fj = jnp.max(jnp.where(lanes == j, fvec, -1))

                    @pl.when(fj >= 0)
                    def _(fj=fj, j=j):
                        pltpu.async_copy(
                            img_hbm.at[pl.ds(fj, 1)], buf.at[pl.ds(j, 1)],
                            fsem)
                for j in range(chunk):
                    fj = jnp.max(jnp.where(lanes == j, fvec, -1))

                    @pl.when(fj >= 0)
                    def _(j=j):
                        pltpu.make_async_copy(
                            img_hbm.at[pl.ds(0, 1)], buf.at[pl.ds(j, 1)],
                            fsem).wait()

        def step(c, r):
            buf, gs, ws = bufs[r]
            nbuf_, ngs, nws = bufs[(r + 2) % 3]
            gather_wait(buf, gs)
            fixup(c, buf)
            pltpu.async_copy(buf, out_hbm.at[pl.ds(base + c, chunk)], ws)
            nxt = c + 2 * chunk

            @pl.when((nxt < per_w) & (nxt >= 3 * chunk))
            def _():
                write_wait(nbuf_, nws)
                gather_start(nxt, nbuf_, ngs)

        n_macro = (per_w // chunk) // 3 * 3

        @pl.loop(0, n_macro * chunk, step=3 * chunk)
        def _(c0):
            for r in range(3):
                step(c0 + r * chunk, r)

        for i in range(n_macro, per_w // chunk):
            step(i * chunk, i % 3)

        write_wait(rows0, wsem0)
        write_wait(rows1, wsem1)
        write_wait(rows2, wsem2)

    return k(ids_flat, table, image)


def kernel(input_ids, image_embeds, embed_weight):
    b, s = input_ids.shape
    d = embed_weight.shape[1]
    out = _sc_combine(input_ids.reshape(-1), embed_weight, image_embeds)
    return out.reshape(b, s, d)


# speedup vs baseline: 1.0012x; 1.0012x over previous
import dataclasses
import functools

import jax
import jax.numpy as jnp
from jax import lax
from jax.experimental import pallas as pl
from jax.experimental.pallas import tpu as pltpu
from jax.experimental.pallas import tpu_sc as plsc

IMG_TOKEN = 151655

NC = 2
NS = 16
NW = NC * NS
LANES = 16


def _sc_combine(ids_flat, table, image):
    n = ids_flat.shape[0]
    d = table.shape[1]
    n_img = image.shape[0]
    per_w = n // NW
    chunk = 16
    mesh = plsc.VectorSubcoreMesh(core_axis_name="c", subcore_axis_name="s")
    cp = pltpu.CompilerParams()
    if "needs_layout_passes" in pltpu.CompilerParams.__dataclass_fields__:
        cp = dataclasses.replace(cp, needs_layout_passes=False)

    @functools.partial(
        pl.kernel,
        out_type=jax.ShapeDtypeStruct((n, d), jnp.float32),
        mesh=mesh,
        compiler_params=cp,
        scratch_types=[
            pltpu.VMEM((n,), jnp.int32),
            pltpu.VMEM((LANES,), jnp.int32),
            pltpu.VMEM((chunk, d), jnp.float32),
            pltpu.VMEM((chunk, d), jnp.float32),
            pltpu.VMEM((chunk, d), jnp.float32),
            pltpu.SemaphoreType.DMA,
            pltpu.SemaphoreType.DMA,
            pltpu.SemaphoreType.DMA,
            pltpu.SemaphoreType.DMA,
            pltpu.SemaphoreType.DMA,
            pltpu.SemaphoreType.DMA,
            pltpu.SemaphoreType.DMA,
        ],
    )
    def k(ids_hbm, table_hbm, img_hbm, out_hbm,
          ids_v, cnt_v, rows0, rows1, rows2,
          gsem0, gsem1, gsem2, wsem0, wsem1, wsem2, fsem):
        wid = lax.axis_index("s") * NC + lax.axis_index("c")
        base = wid * per_w
        pltpu.sync_copy(ids_hbm, ids_v)
        bufs = ((rows0, gsem0, wsem0), (rows1, gsem1, wsem1),
                (rows2, gsem2, wsem2))

        def gather_start(c, buf, gs):
            idv = ids_v[pl.ds(base + c, chunk)]
            pltpu.async_copy(table_hbm.at[idv], buf, gs)

        def gather_wait(buf, gs):
            pltpu.make_async_copy(
                table_hbm.at[ids_v.at[pl.ds(0, chunk)]], buf, gs).wait()

        def write_wait(buf, ws):
            pltpu.make_async_copy(buf, out_hbm.at[pl.ds(0, chunk)], ws).wait()

        gather_start(0, rows0, gsem0)
        gather_start(chunk, rows1, gsem1)
        gather_start(2 * chunk, rows2, gsem2)

        cnt_v[...] = jnp.zeros((LANES,), jnp.int32)

        @pl.loop(0, base, step=LANES)
        def _(g):
            mv = ids_v[pl.ds(g, LANES)] == IMG_TOKEN
            cnt_v[...] = cnt_v[...] + plsc.all_reduce_population_count(mv)

        def fixup(c, buf):
            mvec = ids_v[pl.ds(base + c, chunk)] == IMG_TOKEN
            cs = plsc.cumsum(mvec.astype(jnp.int32))
            tot = jnp.max(cs)
            cntv = cnt_v[...]
            fvec = jnp.where(
                mvec, jnp.clip(cntv + cs - 1, 0, n_img - 1), -1)
            cnt_v[...] = cntv + tot

            @pl.when(tot > 0)
            def _():
                lanes = lax.iota(jnp.int32, chunk)
                for j in range(chunk):
                    fj = jnp.max(jnp.where(lanes == j, fvec, -1))

                    @pl.when(fj >= 0)
                    def _(fj=fj, j=j):
                        pltpu.async_copy(
                            img_hbm.at[pl.ds(fj, 1)], buf.at[pl.ds(j, 1)],
                            fsem)
                for j in range(chunk):
                    fj = jnp.max(jnp.where(lanes == j, fvec, -1))

                    @pl.when(fj >= 0)
                    def _(j=j):
                        pltpu.make_async_copy(
                            img_hbm.at[pl.ds(0, 1)], buf.at[pl.ds(j, 1)],
                            fsem).wait()

        def step(c, r):
            buf, gs, ws = bufs[r]
            nbuf_, ngs, nws = bufs[(r + 2) % 3]
            gather_wait(buf, gs)
            fixup(c, buf)
            pltpu.async_copy(buf, out_hbm.at[pl.ds(base + c, chunk)], ws)
            nxt = c + 2 * chunk

            @pl.when((nxt < per_w) & (nxt >= 3 * chunk))
            def _():
                write_wait(nbuf_, nws)
                gather_start(nxt, nbuf_, ngs)

        n_macro = (per_w // chunk) // 3 * 3

        @pl.loop(0, n_macro * chunk, step=3 * chunk)
        def _(c0):
            for r in range(3):
                step(c0 + r * chunk, r)

        for i in range(n_macro, per_w // chunk):
            step(i * chunk, i % 3)

        write_wait(rows0, wsem0)
        write_wait(rows1, wsem1)
        write_wait(rows2, wsem2)

    return k(ids_flat, table, image)


def kernel(input_ids, image_embeds, embed_weight):
    b, s = input_ids.shape
    d = embed_weight.shape[1]
    out = _sc_combine(input_ids.reshape(-1), embed_weight, image_embeds)
    return out.reshape(b, s, d)


# speedup vs baseline: 1.0037x; 1.0025x over previous
import dataclasses
import functools

import jax
import jax.numpy as jnp
from jax import lax
from jax.experimental import pallas as pl
from jax.experimental.pallas import tpu as pltpu
from jax.experimental.pallas import tpu_sc as plsc

IMG_TOKEN = 151655

NC = 2
NS = 16
NW = NC * NS
LANES = 16


def _sc_combine(ids_flat, table, image):
    n = ids_flat.shape[0]
    d = table.shape[1]
    n_img = image.shape[0]
    per_w = n // NW
    chunk = 16
    mesh = plsc.VectorSubcoreMesh(core_axis_name="c", subcore_axis_name="s")
    cp = pltpu.CompilerParams()
    if "needs_layout_passes" in pltpu.CompilerParams.__dataclass_fields__:
        cp = dataclasses.replace(cp, needs_layout_passes=False)

    @functools.partial(
        pl.kernel,
        out_type=jax.ShapeDtypeStruct((n, d), jnp.float32),
        mesh=mesh,
        compiler_params=cp,
        scratch_types=[
            pltpu.VMEM((n,), jnp.int32),
            pltpu.VMEM((LANES,), jnp.int32),
            pltpu.VMEM((chunk, d), jnp.float32),
            pltpu.VMEM((chunk, d), jnp.float32),
            pltpu.VMEM((chunk, d), jnp.float32),
            pltpu.SemaphoreType.DMA,
            pltpu.SemaphoreType.DMA,
            pltpu.SemaphoreType.DMA,
            pltpu.SemaphoreType.DMA,
            pltpu.SemaphoreType.DMA,
            pltpu.SemaphoreType.DMA,
            pltpu.SemaphoreType.DMA,
        ],
    )
    def k(ids_hbm, table_hbm, img_hbm, out_hbm,
          ids_v, cnt_v, rows0, rows1, rows2,
          gsem0, gsem1, gsem2, wsem0, wsem1, wsem2, fsem):
        wid = lax.axis_index("s") * NC + lax.axis_index("c")
        base = wid * per_w
        pltpu.sync_copy(ids_hbm, ids_v)
        bufs = ((rows0, gsem0, wsem0), (rows1, gsem1, wsem1),
                (rows2, gsem2, wsem2))

        def gather_start(c, buf, gs):
            h = chunk // 2
            pltpu.async_copy(
                table_hbm.at[ids_v.at[pl.ds(base + c, h)]],
                buf.at[pl.ds(0, h)], gs)
            pltpu.async_copy(
                table_hbm.at[ids_v.at[pl.ds(base + c + h, h)]],
                buf.at[pl.ds(h, h)], gs)

        def gather_wait(buf, gs):
            pltpu.make_async_copy(
                table_hbm.at[ids_v.at[pl.ds(0, chunk)]], buf, gs).wait()

        def write_wait(buf, ws):
            pltpu.make_async_copy(buf, out_hbm.at[pl.ds(0, chunk)], ws).wait()

        gather_start(0, rows0, gsem0)
        gather_start(chunk, rows1, gsem1)
        gather_start(2 * chunk, rows2, gsem2)

        cnt_v[...] = jnp.zeros((LANES,), jnp.int32)

        @pl.loop(0, base, step=LANES)
        def _(g):
            mv = ids_v[pl.ds(g, LANES)] == IMG_TOKEN
            cnt_v[...] = cnt_v[...] + plsc.all_reduce_population_count(mv)

        def fixup(c, buf):
            mvec = ids_v[pl.ds(base + c, chunk)] == IMG_TOKEN
            cs = plsc.cumsum(mvec.astype(jnp.int32))
            tot = jnp.max(cs)
            cntv = cnt_v[...]
            fvec = jnp.where(
                mvec, jnp.clip(cntv + cs - 1, 0, n_img - 1), -1)
            cnt_v[...] = cntv + tot

            @pl.when(tot > 0)
            def _():
                lanes = lax.iota(jnp.int32, chunk)
                for j in range(chunk):
                    fj = jnp.max(jnp.where(lanes == j, fvec, -1))

                    @pl.when(fj >= 0)
                    def _(fj=fj, j=j):
                        pltpu.async_copy(
                            img_hbm.at[pl.ds(fj, 1)], buf.at[pl.ds(j, 1)],
                            fsem)
                for j in range(chunk):
                    fj = jnp.max(jnp.where(lanes == j, fvec, -1))

                    @pl.when(fj >= 0)
                    def _(j=j):
                        pltpu.make_async_copy(
                            img_hbm.at[pl.ds(0, 1)], buf.at[pl.ds(j, 1)],
                            fsem).wait()

        def step(c, r):
            buf, gs, ws = bufs[r]
            nbuf_, ngs, nws = bufs[(r + 2) % 3]
            gather_wait(buf, gs)
            fixup(c, buf)
            pltpu.async_copy(buf, out_hbm.at[pl.ds(base + c, chunk)], ws)
            nxt = c + 2 * chunk

            @pl.when((nxt < per_w) & (nxt >= 3 * chunk))
            def _():
                write_wait(nbuf_, nws)
                gather_start(nxt, nbuf_, ngs)

        n_macro = (per_w // chunk) // 3 * 3

        @pl.loop(0, n_macro * chunk, step=3 * chunk)
        def _(c0):
            for r in range(3):
                step(c0 + r * chunk, r)

        for i in range(n_macro, per_w // chunk):
            step(i * chunk, i % 3)

        write_wait(rows0, wsem0)
        write_wait(rows1, wsem1)
        write_wait(rows2, wsem2)

    return k(ids_flat, table, image)


def kernel(input_ids, image_embeds, embed_weight):
    b, s = input_ids.shape
    d = embed_weight.shape[1]
    out = _sc_combine(input_ids.reshape(-1), embed_weight, image_embeds)
    return out.reshape(b, s, d)
